# baseline (speedup 1.0000x reference)
import jax
import jax.numpy as jnp
from jax import lax
from jax.experimental import pallas as pl
from jax.experimental.pallas import tpu as pltpu
from jax.experimental.pallas import tpu_sc as plsc

BATCH = 4096
SEQ = 50
VOCAB = 100000
LANES = 16
NC = 2
NS = 16
NW = NC * NS
ROWS_PER_W = BATCH // NW
IDX_PER_W = ROWS_PER_W * SEQ
CHUNK = 128
NCHUNK = IDX_PER_W // CHUNK
NGROUP = ROWS_PER_W // LANES
HALF_CHUNKS = NCHUNK // 2
HALF_VALS = HALF_CHUNKS * CHUNK


def _body(xf, yf, w, u, bvec_hbm, out,
          idx_x, idx_y, vals_x, vals_y, acc_v, b_v,
          w_s, u_s, sem_io, sem_a, sem_b, sem_c, sem_d):
    sid = lax.axis_index("s")
    wid = sid * NC + lax.axis_index("c")

    @pl.when(sid == 0)
    def _():
        pltpu.sync_copy(w, w_s)

    @pl.when(sid == 1)
    def _():
        pltpu.sync_copy(u, u_s)

    pltpu.async_copy(xf.at[wid], idx_x, sem_io)
    pltpu.async_copy(yf.at[wid], idx_y, sem_io)
    pltpu.async_copy(bvec_hbm, b_v, sem_io)
    pltpu.make_async_copy(xf.at[wid], idx_x, sem_io).wait()
    pltpu.make_async_copy(yf.at[wid], idx_y, sem_io).wait()
    pltpu.make_async_copy(bvec_hbm, b_v, sem_io).wait()

    plsc.subcore_barrier()

    def fire(idx_v, table_s, vals_v, lo, sem):
        def chunk(j, _):
            pltpu.async_copy(
                table_s.at[idx_v.at[j]],
                vals_v.at[pl.ds(j * CHUNK, CHUNK)],
                sem,
            )
            return 0

        lax.fori_loop(lo, lo + HALF_CHUNKS, chunk, 0, unroll=5)

    fire(idx_x, w_s, vals_x, 0, sem_a)
    fire(idx_y, u_s, vals_y, 0, sem_b)
    fire(idx_x, w_s, vals_x, HALF_CHUNKS, sem_c)
    fire(idx_y, u_s, vals_y, HALF_CHUNKS, sem_d)

    def drain(sem):
        pltpu.make_async_copy(
            w.at[pl.ds(0, HALF_VALS)], vals_x.at[pl.ds(0, HALF_VALS)], sem
        ).wait()

    bv = b_v[...]
    zero = jnp.zeros((LANES,), jnp.float32)
    one = jnp.ones((LANES,), jnp.int32)
    rv0 = lax.broadcasted_iota(jnp.int32, (LANES,), 0) * SEQ

    def accum_groups(k_lo, k_hi):
        for k in range(k_lo, k_hi):
            def step(s, carry, base=k * LANES * SEQ):
                ax, ay, pv = carry
                return (ax + plsc.load_gather(vals_x, [pv]),
                        ay + plsc.load_gather(vals_y, [pv]),
                        pv + one)

            ax, ay, _ = lax.fori_loop(
                0, SEQ, step, (zero, zero, rv0 + (k * LANES * SEQ)), unroll=10)
            z = ax + ay + bv
            acc_v[pl.ds(k * LANES, LANES)] = 1.0 / (1.0 + jnp.exp(-z))

    HALF_ROWS = ROWS_PER_W // 2
    out_lo = out.at[pl.ds(wid * ROWS_PER_W, HALF_ROWS)]
    out_hi = out.at[pl.ds(wid * ROWS_PER_W + HALF_ROWS, HALF_ROWS)]
    acc_lo = acc_v.at[pl.ds(0, HALF_ROWS)]
    acc_hi = acc_v.at[pl.ds(HALF_ROWS, HALF_ROWS)]

    drain(sem_a)
    drain(sem_b)
    accum_groups(0, NGROUP // 2)
    pltpu.async_copy(acc_lo, out_lo, sem_io)
    drain(sem_c)
    drain(sem_d)
    accum_groups(NGROUP // 2, NGROUP)
    pltpu.async_copy(acc_hi, out_hi, sem_io)
    pltpu.make_async_copy(acc_lo, out_lo, sem_io).wait()
    pltpu.make_async_copy(acc_hi, out_hi, sem_io).wait()


def kernel(X, Y, W, U, b):
    xf = X.reshape(NW, NCHUNK, CHUNK)
    yf = Y.reshape(NW, NCHUNK, CHUNK)
    bvec = jnp.full((LANES,), b, dtype=jnp.float32)

    mesh = plsc.VectorSubcoreMesh(core_axis_name="c", subcore_axis_name="s")
    f = pl.kernel(
        _body,
        out_type=jax.ShapeDtypeStruct((BATCH,), jnp.float32),
        mesh=mesh,
        compiler_params=pltpu.CompilerParams(needs_layout_passes=False),
        scratch_types=[
            pltpu.VMEM((NCHUNK, CHUNK), jnp.int32),
            pltpu.VMEM((NCHUNK, CHUNK), jnp.int32),
            pltpu.VMEM((IDX_PER_W,), jnp.float32),
            pltpu.VMEM((IDX_PER_W,), jnp.float32),
            pltpu.VMEM((ROWS_PER_W,), jnp.float32),
            pltpu.VMEM((LANES,), jnp.float32),
            pltpu.VMEM_SHARED((VOCAB,), jnp.float32),
            pltpu.VMEM_SHARED((VOCAB,), jnp.float32),
            pltpu.SemaphoreType.DMA,
            pltpu.SemaphoreType.DMA,
            pltpu.SemaphoreType.DMA,
            pltpu.SemaphoreType.DMA,
            pltpu.SemaphoreType.DMA,
        ],
    )
    return f(xf, yf, W, U, bvec)


# speedup vs baseline: 1.0023x; 1.0023x over previous
import jax
import jax.numpy as jnp
from jax import lax
from jax.experimental import pallas as pl
from jax.experimental.pallas import tpu as pltpu
from jax.experimental.pallas import tpu_sc as plsc

BATCH = 4096
SEQ = 50
VOCAB = 100000
LANES = 16
NC = 2
NS = 16
NW = NC * NS
ROWS_PER_W = BATCH // NW
IDX_PER_W = ROWS_PER_W * SEQ
CHUNK = 128
NCHUNK = IDX_PER_W // CHUNK
NGROUP = ROWS_PER_W // LANES
HALF_CHUNKS = NCHUNK // 2
HALF_VALS = HALF_CHUNKS * CHUNK


def _body(xf, yf, w, u, bvec_hbm, out,
          idx_x, idx_y, vals_x, vals_y, acc_v, b_v,
          w_s, u_s, sem_io, sem_a, sem_b, sem_c, sem_d):
    sid = lax.axis_index("s")
    wid = sid * NC + lax.axis_index("c")

    @pl.when(sid == 0)
    def _():
        pltpu.sync_copy(w, w_s)

    @pl.when(sid == 1)
    def _():
        pltpu.sync_copy(u, u_s)

    pltpu.async_copy(xf.at[wid], idx_x, sem_io)
    pltpu.async_copy(yf.at[wid], idx_y, sem_io)
    pltpu.async_copy(bvec_hbm, b_v, sem_io)
    pltpu.make_async_copy(xf.at[wid], idx_x, sem_io).wait()
    pltpu.make_async_copy(yf.at[wid], idx_y, sem_io).wait()
    pltpu.make_async_copy(bvec_hbm, b_v, sem_io).wait()

    plsc.subcore_barrier()

    def fire(idx_v, table_s, vals_v, lo, sem):
        def chunk(j, _):
            pltpu.async_copy(
                table_s.at[idx_v.at[j]],
                vals_v.at[pl.ds(j * CHUNK, CHUNK)],
                sem,
            )
            return 0

        lax.fori_loop(lo, lo + HALF_CHUNKS, chunk, 0, unroll=5)

    fire(idx_x, w_s, vals_x, 0, sem_a)
    fire(idx_y, u_s, vals_y, 0, sem_b)
    fire(idx_x, w_s, vals_x, HALF_CHUNKS, sem_c)
    fire(idx_y, u_s, vals_y, HALF_CHUNKS, sem_d)

    def drain(sem):
        pltpu.make_async_copy(
            w.at[pl.ds(0, HALF_VALS)], vals_x.at[pl.ds(0, HALF_VALS)], sem
        ).wait()

    bv = b_v[...]
    zero = jnp.zeros((LANES,), jnp.float32)
    one = jnp.ones((LANES,), jnp.int32)
    rv0 = lax.broadcasted_iota(jnp.int32, (LANES,), 0) * SEQ

    def accum_groups(k_lo, k_hi):
        for k in range(k_lo, k_hi):
            def step(s, carry, base=k * LANES * SEQ):
                ax, ay, pv = carry
                return (ax + plsc.load_gather(vals_x, [pv]),
                        ay + plsc.load_gather(vals_y, [pv]),
                        pv + one)

            ax, ay, _ = lax.fori_loop(
                0, SEQ, step, (zero, zero, rv0 + (k * LANES * SEQ)), unroll=10)
            z = ax + ay + bv
            acc_v[pl.ds(k * LANES, LANES)] = 1.0 / (1.0 + jnp.exp(-z))

    drain(sem_a)
    drain(sem_b)
    accum_groups(0, NGROUP // 2)
    drain(sem_c)
    drain(sem_d)
    accum_groups(NGROUP // 2, NGROUP)

    pltpu.sync_copy(acc_v, out.at[pl.ds(wid * ROWS_PER_W, ROWS_PER_W)])


def kernel(X, Y, W, U, b):
    xf = X.reshape(NW, NCHUNK, CHUNK)
    yf = Y.reshape(NW, NCHUNK, CHUNK)
    bvec = jnp.full((LANES,), b, dtype=jnp.float32)

    mesh = plsc.VectorSubcoreMesh(core_axis_name="c", subcore_axis_name="s")
    f = pl.kernel(
        _body,
        out_type=jax.ShapeDtypeStruct((BATCH,), jnp.float32),
        mesh=mesh,
        compiler_params=pltpu.CompilerParams(needs_layout_passes=False),
        scratch_types=[
            pltpu.VMEM((NCHUNK, CHUNK), jnp.int32),
            pltpu.VMEM((NCHUNK, CHUNK), jnp.int32),
            pltpu.VMEM((IDX_PER_W,), jnp.float32),
            pltpu.VMEM((IDX_PER_W,), jnp.float32),
            pltpu.VMEM((ROWS_PER_W,), jnp.float32),
            pltpu.VMEM((LANES,), jnp.float32),
            pltpu.VMEM_SHARED((VOCAB,), jnp.float32),
            pltpu.VMEM_SHARED((VOCAB,), jnp.float32),
            pltpu.SemaphoreType.DMA,
            pltpu.SemaphoreType.DMA,
            pltpu.SemaphoreType.DMA,
            pltpu.SemaphoreType.DMA,
            pltpu.SemaphoreType.DMA,
        ],
    )
    return f(xf, yf, W, U, bvec)
